# Initial kernel scaffold; baseline (speedup 1.0000x reference)
import jax
import jax.numpy as jnp
from jax.experimental import pallas as pl


def kernel(x, edge_index, W1, b1, W2, b2):
    raise NotImplementedError("write your pallas kernel here")


# speedup vs baseline: 14.1511x; 14.1511x over previous
import functools

import jax
import jax.numpy as jnp
from jax import lax
from jax.experimental import pallas as pl
from jax.experimental.pallas import tpu as pltpu
from jax.experimental.pallas import tpu_sc as plsc

N = 10000
E = 320000
F_IN = 128
HID = 128
CLS = 64

NPAD = 10240
K = 128
NTILES = 32
STEPS = -(-E // (NTILES * K))
EPAD = NTILES * STEPS * K
RPT = NPAD // 16

_MESH = dict(core_axis_name="c", subcore_axis_name="s")

ROW_BLK = 256
GRID = NPAD // ROW_BLK


def _deg_kernel(dstp, ones16, zeros16):
    @functools.partial(
        pl.kernel,
        out_type=jax.ShapeDtypeStruct((2, NPAD, 16), jnp.float32),
        mesh=plsc.VectorSubcoreMesh(**_MESH),
        compiler_params=pltpu.CompilerParams(use_tc_tiling_on_sc=False),
        scratch_types=[
            pltpu.VMEM((STEPS, K), jnp.int32),
            pltpu.VMEM((K, 16), jnp.float32),
            pltpu.VMEM_SHARED((NPAD, 16), jnp.float32),
        ],
    )
    def body(dst_hbm, ones_hbm, zeros_hbm, out_hbm, dstx, ones_v, acc):
        c = lax.axis_index("c")
        s = lax.axis_index("s")
        wid = s * 2 + c
        r0 = s * RPT
        pltpu.sync_copy(zeros_hbm.at[pl.ds(r0, RPT)], acc.at[pl.ds(r0, RPT)])
        pltpu.sync_copy(ones_hbm, ones_v)
        pltpu.sync_copy(dst_hbm.at[wid], dstx)
        plsc.subcore_barrier()

        def step(j, carry):
            pltpu.sync_copy(ones_v, acc.at[dstx.at[j]], add=True)
            return carry

        lax.fori_loop(0, STEPS, step, 0)
        plsc.subcore_barrier()
        pltpu.sync_copy(acc.at[pl.ds(r0, RPT)], out_hbm.at[c, pl.ds(r0, RPT)])

    return body(dstp, ones16, zeros16)


def _make_spmm(width):
    @functools.partial(
        pl.kernel,
        out_type=jax.ShapeDtypeStruct((2, NPAD, width), jnp.float32),
        mesh=plsc.VectorSubcoreMesh(**_MESH),
        compiler_params=pltpu.CompilerParams(use_tc_tiling_on_sc=False),
        scratch_types=[
            pltpu.VMEM((STEPS, K), jnp.int32),
            pltpu.VMEM((STEPS, K), jnp.int32),
            pltpu.VMEM((K, width), jnp.float32),
            pltpu.VMEM_SHARED((NPAD, width), jnp.float32),
            pltpu.SemaphoreType.DMA,
        ],
    )
    def body(g_hbm, src_hbm, dst_hbm, zeros_hbm, out_hbm, srcx, dstx, buf, acc, sem):
        c = lax.axis_index("c")
        s = lax.axis_index("s")
        wid = s * 2 + c
        r0 = s * RPT
        pltpu.sync_copy(zeros_hbm.at[pl.ds(r0, RPT)], acc.at[pl.ds(r0, RPT)])
        pltpu.sync_copy(src_hbm.at[wid], srcx)
        pltpu.sync_copy(dst_hbm.at[wid], dstx)
        plsc.subcore_barrier()

        def step(j, carry):
            pltpu.async_copy(g_hbm.at[srcx.at[j]], buf, sem).wait()
            pltpu.sync_copy(buf, acc.at[dstx.at[j]], add=True)
            return carry

        lax.fori_loop(0, STEPS, step, 0)
        plsc.subcore_barrier()
        pltpu.sync_copy(acc.at[pl.ds(r0, RPT)], out_hbm.at[c, pl.ds(r0, RPT)])

    return body


_spmm128 = _make_spmm(HID)
_spmm64 = _make_spmm(CLS)


def _deg_dinv(degp):
    deg = 1.0 + jnp.sum(degp[0], axis=1) + jnp.sum(degp[1], axis=1)
    dinv = lax.rsqrt(deg)
    return deg, dinv


def _tc1_body(degp_ref, x_ref, w1_ref, g1_ref):
    _, dinv = _deg_dinv(degp_ref[...])
    g1_ref[...] = jnp.dot(dinv[:, None] * x_ref[...], w1_ref[...],
                          preferred_element_type=jnp.float32)


def _tc2_body(degp_ref, part_ref, g1_ref, w2_ref, b1_ref, g2_ref):
    deg, dinv = _deg_dinv(degp_ref[...])
    p = part_ref[...]
    ssum = p[0] + p[1] + g1_ref[...]
    h1 = jnp.maximum((dinv / deg)[:, None] * ssum + b1_ref[...], 0.0)
    g2_ref[...] = dinv[:, None] * jnp.dot(h1, w2_ref[...],
                                          preferred_element_type=jnp.float32)


def _tc3_body(degp_ref, part_ref, g2_ref, b2_ref, out_ref):
    deg, dinv = _deg_dinv(degp_ref[...])
    p = part_ref[...]
    a = (dinv / deg)[:, None] * (p[0] + p[1] + g2_ref[...]) + b2_ref[...]
    m = jnp.max(a, axis=1, keepdims=True)
    ex = jnp.exp(a - m)
    out_ref[...] = (a - m) - jnp.log(jnp.sum(ex, axis=1, keepdims=True))


def _degp_spec():
    return pl.BlockSpec((2, ROW_BLK, 16), lambda i: (0, i, 0))


def _tc1(degp, xp, W1):
    return pl.pallas_call(
        _tc1_body,
        grid=(GRID,),
        in_specs=[
            _degp_spec(),
            pl.BlockSpec((ROW_BLK, F_IN), lambda i: (i, 0)),
            pl.BlockSpec((F_IN, HID), lambda i: (0, 0)),
        ],
        out_specs=pl.BlockSpec((ROW_BLK, HID), lambda i: (i, 0)),
        out_shape=jax.ShapeDtypeStruct((NPAD, HID), jnp.float32),
    )(degp, xp, W1)


def _tc2(degp, part1, g1, W2, b1):
    return pl.pallas_call(
        _tc2_body,
        grid=(GRID,),
        in_specs=[
            _degp_spec(),
            pl.BlockSpec((2, ROW_BLK, HID), lambda i: (0, i, 0)),
            pl.BlockSpec((ROW_BLK, HID), lambda i: (i, 0)),
            pl.BlockSpec((HID, CLS), lambda i: (0, 0)),
            pl.BlockSpec((1, HID), lambda i: (0, 0)),
        ],
        out_specs=pl.BlockSpec((ROW_BLK, CLS), lambda i: (i, 0)),
        out_shape=jax.ShapeDtypeStruct((NPAD, CLS), jnp.float32),
    )(degp, part1, g1, W2, b1)


def _tc3(degp, part2, g2, b2):
    return pl.pallas_call(
        _tc3_body,
        grid=(GRID,),
        in_specs=[
            _degp_spec(),
            pl.BlockSpec((2, ROW_BLK, CLS), lambda i: (0, i, 0)),
            pl.BlockSpec((ROW_BLK, CLS), lambda i: (i, 0)),
            pl.BlockSpec((1, CLS), lambda i: (0, 0)),
        ],
        out_specs=pl.BlockSpec((ROW_BLK, CLS), lambda i: (i, 0)),
        out_shape=jax.ShapeDtypeStruct((NPAD, CLS), jnp.float32),
    )(degp, part2, g2, b2)


def kernel(x, edge_index, W1, b1, W2, b2):
    pad = EPAD - E
    padv = jnp.full((pad,), N, dtype=jnp.int32)
    srcp = jnp.concatenate([edge_index[0], padv]).reshape(NTILES, STEPS, K)
    dstp = jnp.concatenate([edge_index[1], padv]).reshape(NTILES, STEPS, K)
    xp = jnp.zeros((NPAD, F_IN), jnp.float32).at[:N].set(x)

    ones16 = jnp.ones((K, 16), jnp.float32)
    zeros16 = jnp.zeros((NPAD, 16), jnp.float32)
    zeros128 = jnp.zeros((NPAD, HID), jnp.float32)
    zeros64 = jnp.zeros((NPAD, CLS), jnp.float32)

    degp = _deg_kernel(dstp, ones16, zeros16)
    g1 = _tc1(degp, xp, W1)
    part1 = _spmm128(g1, srcp, dstp, zeros128)
    g2 = _tc2(degp, part1, g1, W2, b1.reshape(1, HID))
    part2 = _spmm64(g2, srcp, dstp, zeros64)
    outp = _tc3(degp, part2, g2, b2.reshape(1, CLS))
    return outp[:N]
